# baseline (device time: 11511 ns/iter reference)
import functools

import jax
import jax.numpy as jnp
from jax import lax
from jax.experimental import pallas as pl
from jax.experimental.pallas import tpu as pltpu

N_DEV = 8
HALO = 3


def kernel(x, k):
    b, s, c = x.shape

    def body(x_ref, k_ref, out_ref, send_ref, recv_ref, send_sem, recv_sem):
        my_i = lax.axis_index("i")
        left = (my_i - 1) % N_DEV
        right = (my_i + 1) % N_DEV

        barrier_sem = pltpu.get_barrier_semaphore()
        for nbr in (left, right):
            pl.semaphore_signal(
                barrier_sem, inc=1,
                device_id=(nbr,), device_id_type=pl.DeviceIdType.MESH,
            )
        pl.semaphore_wait(barrier_sem, 2)

        send_ref[...] = x_ref[:, s - HALO:, :]
        rdma = pltpu.make_async_remote_copy(
            src_ref=send_ref,
            dst_ref=recv_ref,
            send_sem=send_sem,
            recv_sem=recv_sem,
            device_id=(right,),
            device_id_type=pl.DeviceIdType.MESH,
        )

        @pl.when(my_i < N_DEV - 1)
        def _():
            rdma.start()

        @pl.when(my_i == 0)
        def _():
            recv_ref[...] = jnp.zeros((b, HALO, c), jnp.float32)

        @pl.when(my_i > 0)
        def _():
            rdma.wait_recv()

        x_val = x_ref[...]
        halo = recv_ref[...]
        pad = jnp.concatenate([halo, x_val], axis=1)
        acc = pad[:, 0:s, :] * k_ref[0, :]
        acc = acc + pad[:, 1:s + 1, :] * k_ref[1, :]
        acc = acc + pad[:, 2:s + 2, :] * k_ref[2, :]
        acc = acc + x_val * k_ref[3, :]
        out_ref[...] = acc * (1.0 / (1.0 + jnp.exp(-acc)))

        @pl.when(my_i < N_DEV - 1)
        def _():
            rdma.wait_send()

        @functools.partial(pl.run_scoped, exit_sem=pltpu.SemaphoreType.REGULAR)
        def _(exit_sem):
            for nbr in (left, right):
                pl.semaphore_signal(
                    exit_sem, inc=1,
                    device_id=(nbr,), device_id_type=pl.DeviceIdType.MESH,
                )
            pl.semaphore_wait(exit_sem, 2)

    return pl.pallas_call(
        body,
        out_shape=jax.ShapeDtypeStruct((b, s, c), jnp.float32),
        in_specs=[
            pl.BlockSpec(memory_space=pltpu.VMEM),
            pl.BlockSpec(memory_space=pltpu.VMEM),
        ],
        out_specs=pl.BlockSpec(memory_space=pltpu.VMEM),
        scratch_shapes=[
            pltpu.VMEM((b, HALO, c), jnp.float32),
            pltpu.VMEM((b, HALO, c), jnp.float32),
            pltpu.SemaphoreType.DMA,
            pltpu.SemaphoreType.DMA,
        ],
        compiler_params=pltpu.CompilerParams(collective_id=0),
    )(x, k)


# device time: 10796 ns/iter; 1.0662x vs baseline; 1.0662x over previous
import functools

import jax
import jax.numpy as jnp
from jax import lax
from jax.experimental import pallas as pl
from jax.experimental.pallas import tpu as pltpu

N_DEV = 8
HALO = 3


def kernel(x, k):
    b, s, c = x.shape

    def body(x_ref, k_ref, out_ref, send_ref, recv_ref, send_sem, recv_sem):
        my_i = lax.axis_index("i")
        left = (my_i - 1) % N_DEV
        right = (my_i + 1) % N_DEV

        barrier_sem = pltpu.get_barrier_semaphore()
        for nbr in (left, right):
            pl.semaphore_signal(
                barrier_sem, inc=1,
                device_id=(nbr,), device_id_type=pl.DeviceIdType.MESH,
            )
        pl.semaphore_wait(barrier_sem, 2)

        send_ref[...] = x_ref[:, s - HALO:, :]
        rdma = pltpu.make_async_remote_copy(
            src_ref=send_ref,
            dst_ref=recv_ref,
            send_sem=send_sem,
            recv_sem=recv_sem,
            device_id=(right,),
            device_id_type=pl.DeviceIdType.MESH,
        )

        @pl.when(my_i < N_DEV - 1)
        def _():
            rdma.start()

        x_val = x_ref[...]
        zpad = jnp.concatenate(
            [jnp.zeros((b, HALO, c), jnp.float32), x_val], axis=1
        )
        acc = zpad[:, 0:s, :] * k_ref[0, :]
        acc = acc + zpad[:, 1:s + 1, :] * k_ref[1, :]
        acc = acc + zpad[:, 2:s + 2, :] * k_ref[2, :]
        acc = acc + x_val * k_ref[3, :]
        out_ref[...] = acc

        @pl.when(my_i > 0)
        def _():
            rdma.wait_recv()
            h = recv_ref[...]
            hpad = jnp.concatenate(
                [h, jnp.zeros((b, HALO, c), jnp.float32)], axis=1
            )
            fix = hpad[:, 0:HALO, :] * k_ref[0, :]
            fix = fix + hpad[:, 1:HALO + 1, :] * k_ref[1, :]
            fix = fix + hpad[:, 2:HALO + 2, :] * k_ref[2, :]
            out_ref[:, 0:HALO, :] = out_ref[:, 0:HALO, :] + fix

        a = out_ref[...]
        out_ref[...] = a * (1.0 / (1.0 + jnp.exp(-a)))

        @pl.when(my_i < N_DEV - 1)
        def _():
            rdma.wait_send()

        @functools.partial(pl.run_scoped, exit_sem=pltpu.SemaphoreType.REGULAR)
        def _(exit_sem):
            for nbr in (left, right):
                pl.semaphore_signal(
                    exit_sem, inc=1,
                    device_id=(nbr,), device_id_type=pl.DeviceIdType.MESH,
                )
            pl.semaphore_wait(exit_sem, 2)

    return pl.pallas_call(
        body,
        out_shape=jax.ShapeDtypeStruct((b, s, c), jnp.float32),
        in_specs=[
            pl.BlockSpec(memory_space=pltpu.VMEM),
            pl.BlockSpec(memory_space=pltpu.VMEM),
        ],
        out_specs=pl.BlockSpec(memory_space=pltpu.VMEM),
        scratch_shapes=[
            pltpu.VMEM((b, HALO, c), jnp.float32),
            pltpu.VMEM((b, HALO, c), jnp.float32),
            pltpu.SemaphoreType.DMA,
            pltpu.SemaphoreType.DMA,
        ],
        compiler_params=pltpu.CompilerParams(collective_id=0),
    )(x, k)


# device time: 8036 ns/iter; 1.4324x vs baseline; 1.3435x over previous
import jax
import jax.numpy as jnp
from jax import lax
from jax.experimental import pallas as pl
from jax.experimental.pallas import tpu as pltpu

N_DEV = 8
HALO = 3


def kernel(x, k):
    b, s, c = x.shape

    def body(x_ref, k_ref, out_ref, send_ref, recv_ref, send_sem, recv_sem):
        my_i = lax.axis_index("i")
        left = (my_i - 1) % N_DEV
        right = (my_i + 1) % N_DEV

        credit_sem = pltpu.get_barrier_semaphore()

        @pl.when(my_i > 0)
        def _():
            pl.semaphore_signal(
                credit_sem, inc=1,
                device_id=(left,), device_id_type=pl.DeviceIdType.MESH,
            )

        @pl.when(my_i == 0)
        def _():
            recv_ref[...] = jnp.zeros((b, HALO, c), jnp.float32)

        send_ref[...] = x_ref[:, s - HALO:, :]
        rdma = pltpu.make_async_remote_copy(
            src_ref=send_ref,
            dst_ref=recv_ref,
            send_sem=send_sem,
            recv_sem=recv_sem,
            device_id=(right,),
            device_id_type=pl.DeviceIdType.MESH,
        )

        @pl.when(my_i < N_DEV - 1)
        def _():
            pl.semaphore_wait(credit_sem, 1)
            rdma.start()

        x_val = x_ref[...]
        acc = x_val[:, 0:s - HALO, :] * k_ref[0, :]
        acc = acc + x_val[:, 1:s - 2, :] * k_ref[1, :]
        acc = acc + x_val[:, 2:s - 1, :] * k_ref[2, :]
        acc = acc + x_val[:, HALO:, :] * k_ref[3, :]
        out_ref[:, HALO:, :] = acc * (1.0 / (1.0 + jnp.exp(-acc)))

        @pl.when(my_i > 0)
        def _():
            rdma.wait_recv()

        xw = jnp.concatenate([recv_ref[...], x_val[:, 0:HALO, :]], axis=1)
        head = xw[:, 0:HALO, :] * k_ref[0, :]
        head = head + xw[:, 1:HALO + 1, :] * k_ref[1, :]
        head = head + xw[:, 2:HALO + 2, :] * k_ref[2, :]
        head = head + xw[:, HALO:2 * HALO, :] * k_ref[3, :]
        out_ref[:, 0:HALO, :] = head * (1.0 / (1.0 + jnp.exp(-head)))

        @pl.when(my_i < N_DEV - 1)
        def _():
            rdma.wait_send()

    return pl.pallas_call(
        body,
        out_shape=jax.ShapeDtypeStruct((b, s, c), jnp.float32),
        in_specs=[
            pl.BlockSpec(memory_space=pltpu.VMEM),
            pl.BlockSpec(memory_space=pltpu.VMEM),
        ],
        out_specs=pl.BlockSpec(memory_space=pltpu.VMEM),
        scratch_shapes=[
            pltpu.VMEM((b, HALO, c), jnp.float32),
            pltpu.VMEM((b, HALO, c), jnp.float32),
            pltpu.SemaphoreType.DMA,
            pltpu.SemaphoreType.DMA,
        ],
        compiler_params=pltpu.CompilerParams(collective_id=0),
    )(x, k)
